# Initial kernel scaffold; baseline (speedup 1.0000x reference)
import jax
import jax.numpy as jnp
from jax.experimental import pallas as pl


def kernel(inputs, table):
    raise NotImplementedError("write your pallas kernel here")


# speedup vs baseline: 1.2874x; 1.2874x over previous
import functools

import jax
import jax.numpy as jnp
from jax import lax
from jax.experimental import pallas as pl
from jax.experimental.pallas import tpu as pltpu
from jax.experimental.pallas import tpu_sc as plsc

BOOST = 3.0
NUM_WORKERS = 32
CHUNK = 1024
LANES = 16


@functools.partial(jax.jit, static_argnames=("b_total", "dim"))
def _scaled_gather(idx, table, b_total, dim):
    b_per_w = b_total // NUM_WORKERS
    n_chunks = b_per_w // CHUNK
    mesh = plsc.VectorSubcoreMesh(core_axis_name="c", subcore_axis_name="s")

    @functools.partial(
        pl.kernel,
        out_type=jax.ShapeDtypeStruct((b_total, dim), jnp.float32),
        mesh=mesh,
        scratch_types=[
            pltpu.VMEM((b_per_w,), jnp.int32),
            pltpu.VMEM((CHUNK, dim), jnp.float32),
            pltpu.SemaphoreType.DMA,
        ],
        compiler_params=pltpu.CompilerParams(use_tc_tiling_on_sc=False),
    )
    def k(idx_hbm, table_hbm, out_hbm, idx_v, rows_v, sem):
        wid = lax.axis_index("s") * 2 + lax.axis_index("c")
        base = wid * b_per_w
        pltpu.sync_copy(idx_hbm.at[pl.ds(base, b_per_w)], idx_v)

        def chunk_body(g, carry):
            pltpu.async_copy(
                table_hbm.at[idx_v.at[pl.ds(g * CHUNK, CHUNK)]], rows_v, sem
            ).wait()

            def row_body(i, c):
                for half in range(dim // LANES):
                    sl = pl.ds(half * LANES, LANES)
                    rows_v[i, sl] = rows_v[i, sl] * BOOST
                return c

            lax.fori_loop(0, CHUNK, row_body, 0, unroll=4)
            pltpu.sync_copy(rows_v, out_hbm.at[pl.ds(base + g * CHUNK, CHUNK)])
            return carry

        lax.fori_loop(0, n_chunks, chunk_body, 0)

    return k(idx, table)


def kernel(inputs, table):
    b0, s = inputs.shape
    _, dim = table.shape
    idx = inputs.reshape(b0 * s).astype(jnp.int32)
    out = _scaled_gather(idx, table, b0 * s, dim)
    return out.reshape(b0, s, dim)


# speedup vs baseline: 1.3169x; 1.0229x over previous
import functools

import jax
import jax.numpy as jnp
from jax import lax
from jax.experimental import pallas as pl
from jax.experimental.pallas import tpu as pltpu
from jax.experimental.pallas import tpu_sc as plsc

BOOST = 3.0
NUM_WORKERS = 32
CHUNK = 512
NBUF = 4
LOOKAHEAD = 2
LANES = 16


@functools.partial(jax.jit, static_argnames=("b_total", "dim"))
def _scaled_gather(idx, table, b_total, dim):
    b_per_w = b_total // NUM_WORKERS
    n_chunks = b_per_w // CHUNK
    assert n_chunks % NBUF == 0
    mesh = plsc.VectorSubcoreMesh(core_axis_name="c", subcore_axis_name="s")

    @functools.partial(
        pl.kernel,
        out_type=jax.ShapeDtypeStruct((b_total, dim), jnp.float32),
        mesh=mesh,
        scratch_types=[
            pltpu.VMEM((b_per_w,), jnp.int32),
            [pltpu.VMEM((CHUNK, dim), jnp.float32) for _ in range(NBUF)],
            [pltpu.SemaphoreType.DMA for _ in range(NBUF)],
            [pltpu.SemaphoreType.DMA for _ in range(NBUF)],
        ],
        compiler_params=pltpu.CompilerParams(use_tc_tiling_on_sc=False),
    )
    def k(idx_hbm, table_hbm, out_hbm, idx_v, rows, gsem, wsem):
        wid = lax.axis_index("s") * 2 + lax.axis_index("c")
        base = wid * b_per_w
        pltpu.sync_copy(idx_hbm.at[pl.ds(base, b_per_w)], idx_v)

        def gather(ch, b):
            return pltpu.make_async_copy(
                table_hbm.at[idx_v.at[pl.ds(ch * CHUNK, CHUNK)]],
                rows[b], gsem[b],
            )

        def write(ch, b):
            return pltpu.make_async_copy(
                rows[b], out_hbm.at[pl.ds(base + ch * CHUNK, CHUNK)], wsem[b],
            )

        for b in range(LOOKAHEAD):
            gather(b, b).start()

        def outer(o, carry):
            for b in range(NBUF):
                ch = o * NBUF + b
                nb = (b + LOOKAHEAD) % NBUF
                nch = ch + LOOKAHEAD

                @pl.when(nch < n_chunks)
                def _():
                    @pl.when(nch - NBUF >= 0)
                    def _():
                        write(nch - NBUF, nb).wait()
                    gather(nch, nb).start()

                gather(ch, b).wait()

                def row_body(i, c):
                    for half in range(dim // LANES):
                        sl = pl.ds(half * LANES, LANES)
                        rows[b][i, sl] = rows[b][i, sl] * BOOST
                    return c

                lax.fori_loop(0, CHUNK, row_body, 0, unroll=4)
                write(ch, b).start()
            return carry

        lax.fori_loop(0, n_chunks // NBUF, outer, 0)
        for b in range(NBUF):
            write(n_chunks - NBUF + b, b).wait()

    return k(idx, table)


def kernel(inputs, table):
    b0, s = inputs.shape
    _, dim = table.shape
    idx = inputs.reshape(b0 * s).astype(jnp.int32)
    out = _scaled_gather(idx, table, b0 * s, dim)
    return out.reshape(b0, s, dim)
